# Initial kernel scaffold; baseline (speedup 1.0000x reference)
import jax
import jax.numpy as jnp
from jax.experimental import pallas as pl


def kernel(word, context, U, V):
    raise NotImplementedError("write your pallas kernel here")


# speedup vs baseline: 1.4099x; 1.4099x over previous
import functools

import jax
import jax.numpy as jnp
from jax import lax
from jax.experimental import pallas as pl
from jax.experimental.pallas import tpu as pltpu
from jax.experimental.pallas import tpu_sc as plsc

_VOCAB = 100000
_DIM = 128
_BATCH = 4096
_NC = 2
_NS = 16
_L = 16
_NW = _NC * _NS
_BPW = _BATCH // _NW


def _sc_partials(word, context, U, V):
    mesh = plsc.VectorSubcoreMesh(core_axis_name="c", subcore_axis_name="s")

    @functools.partial(
        pl.kernel,
        mesh=mesh,
        out_type=jax.ShapeDtypeStruct((_NW, _L), jnp.float32),
        scratch_types=[
            pltpu.VMEM((_BPW,), jnp.int32),
            pltpu.VMEM((_BPW,), jnp.int32),
            pltpu.VMEM((_BPW, _DIM), jnp.float32),
            pltpu.VMEM((_BPW, _DIM), jnp.float32),
            pltpu.VMEM((_L,), jnp.float32),
            pltpu.SemaphoreType.DMA,
            pltpu.SemaphoreType.DMA,
        ],
    )
    def k(word_hbm, ctx_hbm, u_hbm, v_hbm, out_hbm,
          widx, cidx, urows, vrows, accv, sem_u, sem_v):
        wid = lax.axis_index("s") * _NC + lax.axis_index("c")
        base = wid * _BPW
        pltpu.sync_copy(word_hbm.at[pl.ds(base, _BPW)], widx)
        pltpu.sync_copy(ctx_hbm.at[pl.ds(base, _BPW)], cidx)
        cu = pltpu.async_copy(u_hbm.at[widx], urows, sem_u)
        cv = pltpu.async_copy(v_hbm.at[cidx], vrows, sem_v)
        cu.wait()
        cv.wait()

        def row(i, acc):
            for j in range(_DIM // _L):
                acc = acc + (urows[i, pl.ds(j * _L, _L)]
                             * vrows[i, pl.ds(j * _L, _L)])
            return acc

        accv[...] = lax.fori_loop(0, _BPW, row, jnp.zeros((_L,), jnp.float32))
        pltpu.sync_copy(accv, out_hbm.at[wid])

    return k(word, context, U, V)


def _finalize(partials):
    def body(p_ref, o_ref):
        s = jnp.sum(p_ref[...])
        o_ref[...] = jnp.broadcast_to(jax.nn.log_sigmoid(s), (1, 1))

    return pl.pallas_call(
        body,
        out_shape=jax.ShapeDtypeStruct((1, 1), jnp.float32),
    )(partials)


def kernel(word, context, U, V):
    partials = _sc_partials(word.astype(jnp.int32), context.astype(jnp.int32),
                            U, V)
    return _finalize(partials)
